# baseline (speedup 1.0000x reference)
import jax
import jax.numpy as jnp
from jax.experimental import pallas as pl
from jax.experimental.pallas import tpu as pltpu

_N_REPEATS = 4
_ROWS = 16384
_COLS = 128
_BLOCK_X_ROWS = 256
_THRESH = 1258292
_NAN_TOKEN = -1.0

_K0 = 0
_K1 = 42
_K2 = _K0 ^ _K1 ^ 0x1BD11BDA
_ROT_A = (13, 15, 26, 6)
_ROT_B = (17, 29, 16, 24)


def _mix(a, b, rots):
    for r in rots:
        a = a + b
        b = (b << jnp.uint32(r)) | (b >> jnp.uint32(32 - r))
        b = a ^ b
    return a, b


def _threefry_bits(idx):
    k0 = jnp.uint32(_K0)
    k1 = jnp.uint32(_K1)
    k2 = jnp.uint32(_K2)
    b = idx + k1
    r = jnp.uint32(_ROT_A[0])
    a = b
    b = a ^ ((b << r) | (b >> (jnp.uint32(32) - r)))
    a, b = _mix(a, b, _ROT_A[1:])
    a, b = a + k1, b + (k2 + jnp.uint32(1))
    a, b = _mix(a, b, _ROT_B)
    a, b = a + k2, b + (k0 + jnp.uint32(2))
    a, b = _mix(a, b, _ROT_A)
    a, b = a + k0, b + (k1 + jnp.uint32(3))
    a, b = _mix(a, b, _ROT_B)
    a, b = a + k1, b + (k2 + jnp.uint32(4))
    a, b = _mix(a, b, _ROT_A)
    a, b = a + k2, b + (k0 + jnp.uint32(5))
    return a ^ b


def _masker_body(x_ref, x_out_ref, xv_out_ref):
    i = pl.program_id(0)

    xb = x_ref[...]
    g = jax.lax.broadcasted_iota(jnp.uint32, (_BLOCK_X_ROWS, _COLS), 0)
    c = jax.lax.broadcasted_iota(jnp.uint32, (_BLOCK_X_ROWS, _COLS), 1)
    base0 = jnp.uint32(i) * jnp.uint32(_BLOCK_X_ROWS * _N_REPEATS * _COLS)
    idx0 = base0 + (g << jnp.uint32(9)) + c

    x_out_ref[:, 0:_COLS] = xb
    xv_out_ref[:, 0:_COLS] = xb
    for m in range(1, _N_REPEATS):
        bits = _threefry_bits(idx0 + jnp.uint32(m * _COLS))
        corrupt = bits < jnp.uint32(_THRESH << 9)
        sl = slice(m * _COLS, (m + 1) * _COLS)
        x_out_ref[:, sl] = xb
        xv_out_ref[:, sl] = jnp.where(corrupt, jnp.float32(_NAN_TOKEN), xb)


@jax.jit
def kernel(x):
    grid = (_ROWS // _BLOCK_X_ROWS,)
    wide = _N_REPEATS * _COLS
    X2, XV2 = pl.pallas_call(
        _masker_body,
        grid=grid,
        in_specs=[pl.BlockSpec((_BLOCK_X_ROWS, _COLS), lambda i: (i, 0))],
        out_specs=[
            pl.BlockSpec((_BLOCK_X_ROWS, wide), lambda i: (i, 0)),
            pl.BlockSpec((_BLOCK_X_ROWS, wide), lambda i: (i, 0)),
        ],
        out_shape=[
            jax.ShapeDtypeStruct((_ROWS, wide), jnp.float32),
            jax.ShapeDtypeStruct((_ROWS, wide), jnp.float32),
        ],
        compiler_params=pltpu.CompilerParams(
            dimension_semantics=("parallel",),
        ),
    )(x)
    out_rows = _ROWS * _N_REPEATS
    return (X2.reshape(out_rows, _COLS), XV2.reshape(out_rows, _COLS))


# speedup vs baseline: 1.6387x; 1.6387x over previous
import jax
import jax.numpy as jnp
from jax.experimental import pallas as pl
from jax.experimental.pallas import tpu as pltpu

_N_REPEATS = 4
_ROWS = 16384
_COLS = 128
_BLOCK_X_ROWS = 256
_NUM_BLOCKS = _ROWS // _BLOCK_X_ROWS
_THRESH = 1258292
_NAN_TOKEN = -1.0

_K0 = 0
_K1 = 42
_K2 = _K0 ^ _K1 ^ 0x1BD11BDA
_ROT_A = (13, 15, 26, 6)
_ROT_B = (17, 29, 16, 24)


def _mix(a, b, rots):
    for r in rots:
        a = a + b
        b = (b << jnp.uint32(r)) | (b >> jnp.uint32(32 - r))
        b = a ^ b
    return a, b


def _threefry_bits(idx):
    k0 = jnp.uint32(_K0)
    k1 = jnp.uint32(_K1)
    k2 = jnp.uint32(_K2)
    b = idx + k1
    r = jnp.uint32(_ROT_A[0])
    a = b
    b = a ^ ((b << r) | (b >> (jnp.uint32(32) - r)))
    a, b = _mix(a, b, _ROT_A[1:])
    a, b = a + k1, b + (k2 + jnp.uint32(1))
    a, b = _mix(a, b, _ROT_B)
    a, b = a + k2, b + (k0 + jnp.uint32(2))
    a, b = _mix(a, b, _ROT_A)
    a, b = a + k0, b + (k1 + jnp.uint32(3))
    a, b = _mix(a, b, _ROT_B)
    a, b = a + k1, b + (k2 + jnp.uint32(4))
    a, b = _mix(a, b, _ROT_A)
    a, b = a + k2, b + (k0 + jnp.uint32(5))
    return a ^ b


def _out_copies(i, slot, xob, xvob, xo_hbm, xvo_hbm, sem):
    xo_view = xo_hbm.reshape(_ROWS, _N_REPEATS, _COLS)
    xvo_view = xvo_hbm.reshape(_ROWS, _N_REPEATS, _COLS)
    rows = pl.ds(i * _BLOCK_X_ROWS, _BLOCK_X_ROWS)
    copies = []
    for m in range(_N_REPEATS):
        lanes = pl.ds(m * _COLS, _COLS)
        copies.append(pltpu.make_async_copy(
            xob.at[slot, :, lanes], xo_view.at[rows, m, :], sem.at[slot]))
        copies.append(pltpu.make_async_copy(
            xvob.at[slot, :, lanes], xvo_view.at[rows, m, :], sem.at[slot]))
    return copies


def _masker_body(x_ref, xo_hbm, xvo_hbm, xob, xvob, sem):
    i = pl.program_id(0)
    slot = jax.lax.rem(i, 2)

    @pl.when(i >= 2)
    def _drain():
        for copy in _out_copies(i - 2, slot, xob, xvob, xo_hbm, xvo_hbm, sem):
            copy.wait()

    xb = x_ref[...]
    g = jax.lax.broadcasted_iota(jnp.uint32, (_BLOCK_X_ROWS, _COLS), 0)
    c = jax.lax.broadcasted_iota(jnp.uint32, (_BLOCK_X_ROWS, _COLS), 1)
    base0 = jnp.uint32(i) * jnp.uint32(_BLOCK_X_ROWS * _N_REPEATS * _COLS)
    idx0 = base0 + (g << jnp.uint32(9)) + c

    xob[slot, :, 0:_COLS] = xb
    xvob[slot, :, 0:_COLS] = xb
    for m in range(1, _N_REPEATS):
        bits = _threefry_bits(idx0 + jnp.uint32(m * _COLS))
        corrupt = bits < jnp.uint32(_THRESH << 9)
        sl = slice(m * _COLS, (m + 1) * _COLS)
        xob[slot, :, sl] = xb
        xvob[slot, :, sl] = jnp.where(corrupt, jnp.float32(_NAN_TOKEN), xb)

    for copy in _out_copies(i, slot, xob, xvob, xo_hbm, xvo_hbm, sem):
        copy.start()

    @pl.when(i == _NUM_BLOCKS - 1)
    def _final_drain():
        @pl.when(i >= 1)
        def _other():
            for copy in _out_copies(i - 1, 1 - slot, xob, xvob, xo_hbm,
                                    xvo_hbm, sem):
                copy.wait()
        for copy in _out_copies(i, slot, xob, xvob, xo_hbm, xvo_hbm, sem):
            copy.wait()


@jax.jit
def kernel(x):
    out_rows = _ROWS * _N_REPEATS
    wide = _N_REPEATS * _COLS
    X, XV = pl.pallas_call(
        _masker_body,
        grid=(_NUM_BLOCKS,),
        in_specs=[pl.BlockSpec((_BLOCK_X_ROWS, _COLS), lambda i: (i, 0))],
        out_specs=[
            pl.BlockSpec(memory_space=pltpu.MemorySpace.HBM),
            pl.BlockSpec(memory_space=pltpu.MemorySpace.HBM),
        ],
        out_shape=[
            jax.ShapeDtypeStruct((out_rows, _COLS), jnp.float32),
            jax.ShapeDtypeStruct((out_rows, _COLS), jnp.float32),
        ],
        scratch_shapes=[
            pltpu.VMEM((2, _BLOCK_X_ROWS, wide), jnp.float32),
            pltpu.VMEM((2, _BLOCK_X_ROWS, wide), jnp.float32),
            pltpu.SemaphoreType.DMA((2,)),
        ],
        compiler_params=pltpu.CompilerParams(
            dimension_semantics=("arbitrary",),
        ),
    )(x)
    return (X, XV)


# speedup vs baseline: 1.6688x; 1.0184x over previous
import jax
import jax.numpy as jnp
from jax.experimental import pallas as pl
from jax.experimental.pallas import tpu as pltpu

_N_REPEATS = 4
_ROWS = 16384
_COLS = 128
_BLOCK_X_ROWS = 512
_NUM_BLOCKS = _ROWS // _BLOCK_X_ROWS
_THRESH = 1258292
_NAN_TOKEN = -1.0

_K0 = 0
_K1 = 42
_K2 = _K0 ^ _K1 ^ 0x1BD11BDA
_ROT_A = (13, 15, 26, 6)
_ROT_B = (17, 29, 16, 24)


def _mix(a, b, rots):
    for r in rots:
        a = a + b
        b = (b << jnp.uint32(r)) | (b >> jnp.uint32(32 - r))
        b = a ^ b
    return a, b


def _threefry_bits(idx):
    k0 = jnp.uint32(_K0)
    k1 = jnp.uint32(_K1)
    k2 = jnp.uint32(_K2)
    b = idx + k1
    r = jnp.uint32(_ROT_A[0])
    a = b
    b = a ^ ((b << r) | (b >> (jnp.uint32(32) - r)))
    a, b = _mix(a, b, _ROT_A[1:])
    a, b = a + k1, b + (k2 + jnp.uint32(1))
    a, b = _mix(a, b, _ROT_B)
    a, b = a + k2, b + (k0 + jnp.uint32(2))
    a, b = _mix(a, b, _ROT_A)
    a, b = a + k0, b + (k1 + jnp.uint32(3))
    a, b = _mix(a, b, _ROT_B)
    a, b = a + k1, b + (k2 + jnp.uint32(4))
    a, b = _mix(a, b, _ROT_A)
    a, b = a + k2, b + (k0 + jnp.uint32(5))
    return a ^ b


def _out_copies(i, slot, xob, xvob, xo_hbm, xvo_hbm, sem):
    xo_view = xo_hbm.reshape(_ROWS, _N_REPEATS, _COLS)
    xvo_view = xvo_hbm.reshape(_ROWS, _N_REPEATS, _COLS)
    rows = pl.ds(i * _BLOCK_X_ROWS, _BLOCK_X_ROWS)
    copies = []
    for m in range(_N_REPEATS):
        lanes = pl.ds(m * _COLS, _COLS)
        copies.append(pltpu.make_async_copy(
            xob.at[slot, :, lanes], xo_view.at[rows, m, :], sem.at[slot]))
        copies.append(pltpu.make_async_copy(
            xvob.at[slot, :, lanes], xvo_view.at[rows, m, :], sem.at[slot]))
    return copies


def _masker_body(x_ref, xo_hbm, xvo_hbm, xob, xvob, sem):
    i = pl.program_id(0)
    slot = jax.lax.rem(i, 2)

    @pl.when(i >= 2)
    def _drain():
        for copy in _out_copies(i - 2, slot, xob, xvob, xo_hbm, xvo_hbm, sem):
            copy.wait()

    xb = x_ref[...]
    g = jax.lax.broadcasted_iota(jnp.uint32, (_BLOCK_X_ROWS, _COLS), 0)
    c = jax.lax.broadcasted_iota(jnp.uint32, (_BLOCK_X_ROWS, _COLS), 1)
    base0 = jnp.uint32(i) * jnp.uint32(_BLOCK_X_ROWS * _N_REPEATS * _COLS)
    idx0 = base0 + (g << jnp.uint32(9)) + c

    xob[slot, :, 0:_COLS] = xb
    xvob[slot, :, 0:_COLS] = xb
    for m in range(1, _N_REPEATS):
        bits = _threefry_bits(idx0 + jnp.uint32(m * _COLS))
        corrupt = bits < jnp.uint32(_THRESH << 9)
        sl = slice(m * _COLS, (m + 1) * _COLS)
        xob[slot, :, sl] = xb
        xvob[slot, :, sl] = jnp.where(corrupt, jnp.float32(_NAN_TOKEN), xb)

    for copy in _out_copies(i, slot, xob, xvob, xo_hbm, xvo_hbm, sem):
        copy.start()

    @pl.when(i == _NUM_BLOCKS - 1)
    def _final_drain():
        @pl.when(i >= 1)
        def _other():
            for copy in _out_copies(i - 1, 1 - slot, xob, xvob, xo_hbm,
                                    xvo_hbm, sem):
                copy.wait()
        for copy in _out_copies(i, slot, xob, xvob, xo_hbm, xvo_hbm, sem):
            copy.wait()


@jax.jit
def kernel(x):
    out_rows = _ROWS * _N_REPEATS
    wide = _N_REPEATS * _COLS
    X, XV = pl.pallas_call(
        _masker_body,
        grid=(_NUM_BLOCKS,),
        in_specs=[pl.BlockSpec((_BLOCK_X_ROWS, _COLS), lambda i: (i, 0))],
        out_specs=[
            pl.BlockSpec(memory_space=pltpu.MemorySpace.HBM),
            pl.BlockSpec(memory_space=pltpu.MemorySpace.HBM),
        ],
        out_shape=[
            jax.ShapeDtypeStruct((out_rows, _COLS), jnp.float32),
            jax.ShapeDtypeStruct((out_rows, _COLS), jnp.float32),
        ],
        scratch_shapes=[
            pltpu.VMEM((2, _BLOCK_X_ROWS, wide), jnp.float32),
            pltpu.VMEM((2, _BLOCK_X_ROWS, wide), jnp.float32),
            pltpu.SemaphoreType.DMA((2,)),
        ],
        compiler_params=pltpu.CompilerParams(
            dimension_semantics=("arbitrary",),
        ),
    )(x)
    return (X, XV)


# speedup vs baseline: 1.6716x; 1.0017x over previous
import jax
import jax.numpy as jnp
from jax.experimental import pallas as pl
from jax.experimental.pallas import tpu as pltpu

_N_REPEATS = 4
_ROWS = 16384
_COLS = 128
_BLOCK_X_ROWS = 1024
_NUM_BLOCKS = _ROWS // _BLOCK_X_ROWS
_THRESH = 1258292
_NAN_TOKEN = -1.0

_K0 = 0
_K1 = 42
_K2 = _K0 ^ _K1 ^ 0x1BD11BDA
_ROT_A = (13, 15, 26, 6)
_ROT_B = (17, 29, 16, 24)


def _mix(a, b, rots):
    for r in rots:
        a = a + b
        b = (b << jnp.uint32(r)) | (b >> jnp.uint32(32 - r))
        b = a ^ b
    return a, b


def _threefry_bits(idx):
    k0 = jnp.uint32(_K0)
    k1 = jnp.uint32(_K1)
    k2 = jnp.uint32(_K2)
    b = idx + k1
    r = jnp.uint32(_ROT_A[0])
    a = b
    b = a ^ ((b << r) | (b >> (jnp.uint32(32) - r)))
    a, b = _mix(a, b, _ROT_A[1:])
    a, b = a + k1, b + (k2 + jnp.uint32(1))
    a, b = _mix(a, b, _ROT_B)
    a, b = a + k2, b + (k0 + jnp.uint32(2))
    a, b = _mix(a, b, _ROT_A)
    a, b = a + k0, b + (k1 + jnp.uint32(3))
    a, b = _mix(a, b, _ROT_B)
    a, b = a + k1, b + (k2 + jnp.uint32(4))
    a, b = _mix(a, b, _ROT_A)
    a, b = a + k2, b + (k0 + jnp.uint32(5))
    return a ^ b


def _out_copies(i, slot, xob, xvob, xo_hbm, xvo_hbm, sem):
    xo_view = xo_hbm.reshape(_ROWS, _N_REPEATS, _COLS)
    xvo_view = xvo_hbm.reshape(_ROWS, _N_REPEATS, _COLS)
    rows = pl.ds(i * _BLOCK_X_ROWS, _BLOCK_X_ROWS)
    copies = []
    for m in range(_N_REPEATS):
        lanes = pl.ds(m * _COLS, _COLS)
        copies.append(pltpu.make_async_copy(
            xob.at[slot, :, lanes], xo_view.at[rows, m, :], sem.at[slot]))
        copies.append(pltpu.make_async_copy(
            xvob.at[slot, :, lanes], xvo_view.at[rows, m, :], sem.at[slot]))
    return copies


def _masker_body(x_ref, xo_hbm, xvo_hbm, xob, xvob, sem):
    i = pl.program_id(0)
    slot = jax.lax.rem(i, 2)

    @pl.when(i >= 2)
    def _drain():
        for copy in _out_copies(i - 2, slot, xob, xvob, xo_hbm, xvo_hbm, sem):
            copy.wait()

    xb = x_ref[...]
    g = jax.lax.broadcasted_iota(jnp.uint32, (_BLOCK_X_ROWS, _COLS), 0)
    c = jax.lax.broadcasted_iota(jnp.uint32, (_BLOCK_X_ROWS, _COLS), 1)
    base0 = jnp.uint32(i) * jnp.uint32(_BLOCK_X_ROWS * _N_REPEATS * _COLS)
    idx0 = base0 + (g << jnp.uint32(9)) + c

    xob[slot, :, 0:_COLS] = xb
    xvob[slot, :, 0:_COLS] = xb
    for m in range(1, _N_REPEATS):
        bits = _threefry_bits(idx0 + jnp.uint32(m * _COLS))
        corrupt = bits < jnp.uint32(_THRESH << 9)
        sl = slice(m * _COLS, (m + 1) * _COLS)
        xob[slot, :, sl] = xb
        xvob[slot, :, sl] = jnp.where(corrupt, jnp.float32(_NAN_TOKEN), xb)

    for copy in _out_copies(i, slot, xob, xvob, xo_hbm, xvo_hbm, sem):
        copy.start()

    @pl.when(i == _NUM_BLOCKS - 1)
    def _final_drain():
        @pl.when(i >= 1)
        def _other():
            for copy in _out_copies(i - 1, 1 - slot, xob, xvob, xo_hbm,
                                    xvo_hbm, sem):
                copy.wait()
        for copy in _out_copies(i, slot, xob, xvob, xo_hbm, xvo_hbm, sem):
            copy.wait()


@jax.jit
def kernel(x):
    out_rows = _ROWS * _N_REPEATS
    wide = _N_REPEATS * _COLS
    X, XV = pl.pallas_call(
        _masker_body,
        grid=(_NUM_BLOCKS,),
        in_specs=[pl.BlockSpec((_BLOCK_X_ROWS, _COLS), lambda i: (i, 0))],
        out_specs=[
            pl.BlockSpec(memory_space=pltpu.MemorySpace.HBM),
            pl.BlockSpec(memory_space=pltpu.MemorySpace.HBM),
        ],
        out_shape=[
            jax.ShapeDtypeStruct((out_rows, _COLS), jnp.float32),
            jax.ShapeDtypeStruct((out_rows, _COLS), jnp.float32),
        ],
        scratch_shapes=[
            pltpu.VMEM((2, _BLOCK_X_ROWS, wide), jnp.float32),
            pltpu.VMEM((2, _BLOCK_X_ROWS, wide), jnp.float32),
            pltpu.SemaphoreType.DMA((2,)),
        ],
        compiler_params=pltpu.CompilerParams(
            dimension_semantics=("arbitrary",),
        ),
    )(x)
    return (X, XV)


# speedup vs baseline: 1.7149x; 1.0259x over previous
import jax
import jax.numpy as jnp
from jax.experimental import pallas as pl
from jax.experimental.pallas import tpu as pltpu

_N_REPEATS = 4
_ROWS = 16384
_COLS = 128
_BLOCK_X_ROWS = 1024
_NUM_BLOCKS = _ROWS // _BLOCK_X_ROWS
_THRESH = 1258292
_NAN_TOKEN = -1.0

_K0 = 0
_K1 = 42
_K2 = _K0 ^ _K1 ^ 0x1BD11BDA
_ROT_A = (13, 15, 26, 6)
_ROT_B = (17, 29, 16, 24)


def _mix(a, b, rots):
    for r in rots:
        a = a + b
        b = (b << jnp.uint32(r)) | (b >> jnp.uint32(32 - r))
        b = a ^ b
    return a, b


def _threefry_bits(idx):
    k0 = jnp.uint32(_K0)
    k1 = jnp.uint32(_K1)
    k2 = jnp.uint32(_K2)
    b = idx + k1
    r = jnp.uint32(_ROT_A[0])
    a = b
    b = a ^ ((b << r) | (b >> (jnp.uint32(32) - r)))
    a, b = _mix(a, b, _ROT_A[1:])
    a, b = a + k1, b + (k2 + jnp.uint32(1))
    a, b = _mix(a, b, _ROT_B)
    a, b = a + k2, b + (k0 + jnp.uint32(2))
    a, b = _mix(a, b, _ROT_A)
    a, b = a + k0, b + (k1 + jnp.uint32(3))
    a, b = _mix(a, b, _ROT_B)
    a, b = a + k1, b + (k2 + jnp.uint32(4))
    a, b = _mix(a, b, _ROT_A)
    a, b = a + k2, b + (k0 + jnp.uint32(5))
    return a ^ b


_NSLOTS = 3


def _out_copies(i, slot, xob, xvob, xo_hbm, xvo_hbm, sem):
    xo_view = xo_hbm.reshape(_ROWS, _N_REPEATS, _COLS)
    xvo_view = xvo_hbm.reshape(_ROWS, _N_REPEATS, _COLS)
    rows = pl.ds(i * _BLOCK_X_ROWS, _BLOCK_X_ROWS)
    xsrc = xob.at[slot]
    copies = []
    for m in range(_N_REPEATS):
        copies.append(pltpu.make_async_copy(
            xsrc, xo_view.at[rows, m, :], sem.at[slot]))
    copies.append(pltpu.make_async_copy(
        xsrc, xvo_view.at[rows, 0, :], sem.at[slot]))
    for m in range(1, _N_REPEATS):
        lanes = pl.ds((m - 1) * _COLS, _COLS)
        copies.append(pltpu.make_async_copy(
            xvob.at[slot, :, lanes], xvo_view.at[rows, m, :], sem.at[slot]))
    return copies


def _masker_body(x_ref, xo_hbm, xvo_hbm, xob, xvob, sem):
    i = pl.program_id(0)
    slot = jax.lax.rem(i, _NSLOTS)

    @pl.when(i >= _NSLOTS)
    def _drain():
        for copy in _out_copies(i - _NSLOTS, slot, xob, xvob, xo_hbm,
                                xvo_hbm, sem):
            copy.wait()

    xb = x_ref[...]
    g = jax.lax.broadcasted_iota(jnp.uint32, (_BLOCK_X_ROWS, _COLS), 0)
    c = jax.lax.broadcasted_iota(jnp.uint32, (_BLOCK_X_ROWS, _COLS), 1)
    base0 = jnp.uint32(i) * jnp.uint32(_BLOCK_X_ROWS * _N_REPEATS * _COLS)
    idx0 = base0 + (g << jnp.uint32(9)) + c

    xob[slot] = xb
    for m in range(1, _N_REPEATS):
        bits = _threefry_bits(idx0 + jnp.uint32(m * _COLS))
        corrupt = bits < jnp.uint32(_THRESH << 9)
        sl = slice((m - 1) * _COLS, m * _COLS)
        xvob[slot, :, sl] = jnp.where(corrupt, jnp.float32(_NAN_TOKEN), xb)

    for copy in _out_copies(i, slot, xob, xvob, xo_hbm, xvo_hbm, sem):
        copy.start()

    @pl.when(i == _NUM_BLOCKS - 1)
    def _final_drain():
        for back in range(_NSLOTS - 1, 0, -1):
            @pl.when(i >= back)
            def _prev(back=back):
                prev_slot = jax.lax.rem(i - back + _NSLOTS, _NSLOTS)
                for copy in _out_copies(i - back, prev_slot, xob, xvob,
                                        xo_hbm, xvo_hbm, sem):
                    copy.wait()
        for copy in _out_copies(i, slot, xob, xvob, xo_hbm, xvo_hbm, sem):
            copy.wait()


@jax.jit
def kernel(x):
    out_rows = _ROWS * _N_REPEATS

    X, XV = pl.pallas_call(
        _masker_body,
        grid=(_NUM_BLOCKS,),
        in_specs=[pl.BlockSpec((_BLOCK_X_ROWS, _COLS), lambda i: (i, 0))],
        out_specs=[
            pl.BlockSpec(memory_space=pltpu.MemorySpace.HBM),
            pl.BlockSpec(memory_space=pltpu.MemorySpace.HBM),
        ],
        out_shape=[
            jax.ShapeDtypeStruct((out_rows, _COLS), jnp.float32),
            jax.ShapeDtypeStruct((out_rows, _COLS), jnp.float32),
        ],
        scratch_shapes=[
            pltpu.VMEM((_NSLOTS, _BLOCK_X_ROWS, _COLS), jnp.float32),
            pltpu.VMEM((_NSLOTS, _BLOCK_X_ROWS,
                        (_N_REPEATS - 1) * _COLS), jnp.float32),
            pltpu.SemaphoreType.DMA((_NSLOTS,)),
        ],
        compiler_params=pltpu.CompilerParams(
            dimension_semantics=("arbitrary",),
        ),
    )(x)
    return (X, XV)
